# Initial kernel scaffold; baseline (speedup 1.0000x reference)
import jax
import jax.numpy as jnp
from jax.experimental import pallas as pl


def kernel(node_types, edge_index, edge_type, ptr, emb_table, msg_W, msg_b, gru_w_ih, gru_w_hh, gru_b_ih, gru_b_hh, gate_w, gate_b, n2g_w, n2g_b, loc_w, loc_b, logvar_w, logvar_b):
    raise NotImplementedError("write your pallas kernel here")


# speedup vs baseline: 4.5446x; 4.5446x over previous
import functools

import jax
import jax.numpy as jnp
from jax import lax
from jax.experimental import pallas as pl
from jax.experimental.pallas import tpu as pltpu
from jax.experimental.pallas import tpu_sc as plsc

N = 10000
E = 320000
D = 128
MH = 2 * D
T = 3
NT = 4
G = 256
LT = 64
V = 128
B = 100

NC, NS, LANES = 2, 16, 16
HALF = N // 2
PLANE = 5120
SPR = 2 * PLANE
SLAB = SPR // NS
WCH = 80
EC = 80
EPT = E // NS
NCHUNK = EPT // EC
DUMMY = HALF
RB = 1000

_SC_MESH = dict(core_axis_name="c", subcore_axis_name="s",
                num_cores=NC, num_subcores=NS)


def _make_sc_scatter(ones_mode):
    out_type = jax.ShapeDtypeStruct((NC * 2 * SPR, D), jnp.float32)
    scratch = [
        pltpu.VMEM_SHARED((SPR, D), jnp.float32),
        pltpu.VMEM((EC,), jnp.int32),
        pltpu.VMEM((EC,), jnp.int32),
        pltpu.VMEM((EC,), jnp.int32),
        pltpu.VMEM((EC, D), jnp.float32),
        pltpu.SemaphoreType.DMA,
    ]

    def body(h_hbm, src_hbm, dst_hbm, typ_hbm, zrow_hbm, ones_hbm,
             acc_out, acc_sh, src_v, dst_v, typ_v, rows_v, sem):
        cid = lax.axis_index("c")
        sid = lax.axis_index("s")
        estart = sid * EPT
        slab0 = sid * SLAB

        for p in range(2):
            pltpu.sync_copy(zrow_hbm, rows_v)
            for k in range(SLAB // WCH):
                r0 = slab0 + k * WCH
                pltpu.sync_copy(rows_v, acc_sh.at[pl.ds(r0, WCH), :])
            plsc.subcore_barrier()
            if ones_mode:
                pltpu.sync_copy(ones_hbm, rows_v)

            cidv = jnp.full((LANES,), cid, jnp.int32)
            onev = jnp.full((LANES,), 1, jnp.int32)
            planev = jnp.full((LANES,), PLANE, jnp.int32)
            dummyv = jnp.full((LANES,), DUMMY, jnp.int32)
            lov = jnp.full((LANES,), p * HALF, jnp.int32)
            hiv = jnp.full((LANES,), (p + 1) * HALF, jnp.int32)

            def chunk(i, carry):
                base = pl.multiple_of(estart + i * EC, 8)
                pltpu.sync_copy(src_hbm.at[pl.ds(base, EC)], src_v)
                pltpu.sync_copy(dst_hbm.at[pl.ds(base, EC)], dst_v)
                pltpu.sync_copy(typ_hbm.at[pl.ds(base, EC)], typ_v)
                if not ones_mode:
                    pltpu.async_copy(h_hbm.at[src_v], rows_v, sem).wait()
                for j in range(EC // LANES):
                    t16 = typ_v[pl.ds(j * LANES, LANES)]
                    d16 = dst_v[pl.ds(j * LANES, LANES)]
                    plane_t = lax.shift_right_logical(t16, onev)
                    low = jnp.bitwise_and(t16, onev)
                    owned = (plane_t == cidv) & (d16 >= lov) & (d16 < hiv)
                    q = jnp.where(owned, low * planev + (d16 - lov), dummyv)
                    pltpu.sync_copy(rows_v.at[pl.ds(j * LANES, LANES), :],
                                    acc_sh.at[q], add=True)
                return carry

            lax.fori_loop(0, NCHUNK, chunk, 0)
            plsc.subcore_barrier()

            obase = (cid * 2 + p) * SPR
            for k in range(SLAB // WCH):
                r0 = slab0 + k * WCH
                pltpu.sync_copy(acc_sh.at[pl.ds(r0, WCH), :], rows_v)
                pltpu.sync_copy(rows_v,
                                acc_out.at[pl.ds(obase + r0, WCH), :])

    return pl.kernel(
        body,
        out_type=out_type,
        mesh=plsc.VectorSubcoreMesh(**_SC_MESH),
        scratch_types=scratch,
        name="sc_count" if ones_mode else "sc_scatter",
    )


_sc_scatter_fn = functools.lru_cache(maxsize=None)(_make_sc_scatter)


def _embed_body(nt_ref, emb_ref, out_ref):
    onehot = (nt_ref[...] == lax.broadcasted_iota(jnp.int32, (1, V), 1))
    out_ref[...] = jnp.dot(onehot.astype(jnp.float32), emb_ref[...],
                           preferred_element_type=jnp.float32)


_embed = pl.pallas_call(
    _embed_body,
    grid=(N // RB,),
    in_specs=[
        pl.BlockSpec((RB, 1), lambda i: (i, 0)),
        pl.BlockSpec((V, D), lambda i: (0, 0)),
    ],
    out_specs=pl.BlockSpec((RB, D), lambda i: (i, 0)),
    out_shape=jax.ShapeDtypeStruct((N, D), jnp.float32),
)


def _dotT(a, b):
    return lax.dot_general(a, b, (((1,), (1,)), ((), ())),
                           preferred_element_type=jnp.float32)


def _round_body(h_ref, acc_ref, cnt_ref, mw_ref, mb_ref, wih_ref, whh_ref,
                bih_ref, bhh_ref, out_ref):
    h = h_ref[...]
    total = jnp.zeros((RB, MH), jnp.float32)
    for c in range(NC):
        for pz in range(2):
            t = 2 * c + pz
            total = total + _dotT(acc_ref[c, 0, pz], mw_ref[t])
            total = total + cnt_ref[c, 0, pz][:, 0:1] * mb_ref[t]
    msgs = jnp.maximum(total, 0.0)
    gi = _dotT(msgs, wih_ref[...]) + bih_ref[...]
    gh = _dotT(h, whh_ref[...]) + bhh_ref[...]
    r = jax.nn.sigmoid(gi[:, :D] + gh[:, :D])
    z = jax.nn.sigmoid(gi[:, D:2 * D] + gh[:, D:2 * D])
    cbar = jnp.tanh(gi[:, 2 * D:] + r * gh[:, 2 * D:])
    out_ref[...] = (1.0 - z) * cbar + z * h


_round_tc = pl.pallas_call(
    _round_body,
    grid=(N // RB,),
    in_specs=[
        pl.BlockSpec((RB, D), lambda i: (i, 0)),
        pl.BlockSpec((NC, 1, 2, RB, D), lambda i: (0, i // 5, 0, i % 5, 0)),
        pl.BlockSpec((NC, 1, 2, RB, D), lambda i: (0, i // 5, 0, i % 5, 0)),
        pl.BlockSpec((NT, MH, D), lambda i: (0, 0, 0)),
        pl.BlockSpec((NT, 1, MH), lambda i: (0, 0, 0)),
        pl.BlockSpec((3 * D, MH), lambda i: (0, 0)),
        pl.BlockSpec((3 * D, D), lambda i: (0, 0)),
        pl.BlockSpec((1, 3 * D), lambda i: (0, 0)),
        pl.BlockSpec((1, 3 * D), lambda i: (0, 0)),
    ],
    out_specs=pl.BlockSpec((RB, D), lambda i: (i, 0)),
    out_shape=jax.ShapeDtypeStruct((N, D), jnp.float32),
)


def _readout_body(h_ref, ptr_ref, gw_ref, gb_ref, nw_ref, nb_ref, lw_ref,
                  lb_ref, vw_ref, vb_ref, zl_ref, zv_ref, hg):
    i = pl.program_id(0)
    h = h_ref[...]
    attn = jax.nn.sigmoid(_dotT(gw_ref[...], h) + gb_ref[...])
    h2 = _dotT(h, nw_ref[...]) + nb_ref[...]
    g = i * RB + lax.broadcasted_iota(jnp.int32, (1, RB), 1)
    seg = jnp.sum((ptr_ref[...] <= g).astype(jnp.int32), axis=0,
                  keepdims=True)
    oneT = (lax.broadcasted_iota(jnp.int32, (V, 1), 0) == seg)
    aw = oneT.astype(jnp.float32) * attn
    contrib = jnp.dot(aw, h2, preferred_element_type=jnp.float32)

    @pl.when(i == 0)
    def _():
        hg[...] = jnp.zeros_like(hg)

    hg[...] += contrib

    @pl.when(i == pl.num_programs(0) - 1)
    def _():
        zl = _dotT(hg[...], lw_ref[...]) + lb_ref[...]
        zv = _dotT(hg[...], vw_ref[...]) + vb_ref[...]
        zl_ref[...] = zl[:B, :]
        zv_ref[...] = zv[:B, :]


_readout = pl.pallas_call(
    _readout_body,
    grid=(N // RB,),
    in_specs=[
        pl.BlockSpec((RB, D), lambda i: (i, 0)),
        pl.BlockSpec((V, 1), lambda i: (0, 0)),
        pl.BlockSpec((1, D), lambda i: (0, 0)),
        pl.BlockSpec((1, RB), lambda i: (0, 0)),
        pl.BlockSpec((G, D), lambda i: (0, 0)),
        pl.BlockSpec((1, G), lambda i: (0, 0)),
        pl.BlockSpec((LT, G), lambda i: (0, 0)),
        pl.BlockSpec((1, LT), lambda i: (0, 0)),
        pl.BlockSpec((LT, G), lambda i: (0, 0)),
        pl.BlockSpec((1, LT), lambda i: (0, 0)),
    ],
    out_specs=[
        pl.BlockSpec((B, LT), lambda i: (0, 0)),
        pl.BlockSpec((B, LT), lambda i: (0, 0)),
    ],
    out_shape=[
        jax.ShapeDtypeStruct((B, LT), jnp.float32),
        jax.ShapeDtypeStruct((B, LT), jnp.float32),
    ],
    scratch_shapes=[pltpu.VMEM((V, G), jnp.float32)],
)


def kernel(node_types, edge_index, edge_type, ptr, emb_table, msg_W, msg_b,
           gru_w_ih, gru_w_hh, gru_b_ih, gru_b_hh, gate_w, gate_b, n2g_w,
           n2g_b, loc_w, loc_b, logvar_w, logvar_b):
    nt_col = node_types.astype(jnp.int32).reshape(N, 1)
    dst = edge_index[0].astype(jnp.int32)
    src = edge_index[1].astype(jnp.int32)
    typ = edge_type.astype(jnp.int32)
    ptr_pad = jnp.concatenate(
        [ptr[1:B].astype(jnp.int32),
         jnp.full((V - (B - 1),), N, jnp.int32)]).reshape(V, 1)
    mb3 = msg_b.reshape(NT, 1, MH)
    bih = gru_b_ih.reshape(1, 3 * D)
    bhh = gru_b_hh.reshape(1, 3 * D)
    gb = jnp.full((1, RB), gate_b[0], jnp.float32)
    nb = n2g_b.reshape(1, G)
    lb = loc_b.reshape(1, LT)
    vb = logvar_b.reshape(1, LT)
    zrow = jnp.zeros((WCH, D), jnp.float32)
    ones = jnp.ones((EC, D), jnp.float32)

    h = _embed(nt_col, emb_table)
    cnt = _sc_scatter_fn(True)(h, src, dst, typ, zrow, ones)
    cnt5 = cnt.reshape(NC, 2, 2, PLANE, D)
    for rnd in range(T):
        acc = _sc_scatter_fn(False)(h, src, dst, typ, zrow, ones)
        acc5 = acc.reshape(NC, 2, 2, PLANE, D)
        h = _round_tc(h, acc5, cnt5, msg_W, mb3, gru_w_ih, gru_w_hh, bih, bhh)
    return _readout(h, ptr_pad, gate_w, gb, n2g_w, nb, loc_w, lb,
                    logvar_w, vb)


# speedup vs baseline: 6.9065x; 1.5197x over previous
import functools

import jax
import jax.numpy as jnp
from jax import lax
from jax.experimental import pallas as pl
from jax.experimental.pallas import tpu as pltpu
from jax.experimental.pallas import tpu_sc as plsc

N = 10000
E = 320000
D = 128
MH = 2 * D
T = 3
NT = 4
G = 256
LT = 64
V = 128
B = 100

NC, NS, LANES = 2, 16, 16
HALF = N // 2
PLANE = 5120
SPR = 2 * PLANE
SLAB = SPR // NS
WCH = 80
EC = 80
EPT = E // NS
NCHUNK = EPT // EC
DUMMY = HALF
RB = 1000

_SC_MESH = dict(core_axis_name="c", subcore_axis_name="s",
                num_cores=NC, num_subcores=NS)


def _make_sc_scatter(ones_mode):
    out_type = jax.ShapeDtypeStruct((NC * 2 * SPR, D), jnp.float32)
    scratch = [
        pltpu.VMEM_SHARED((SPR, D), jnp.float32),
        pltpu.VMEM((2 * EC,), jnp.int32),
        pltpu.VMEM((EC,), jnp.int32),
        pltpu.VMEM((EC, D), jnp.float32),
        pltpu.SemaphoreType.DMA,
    ]

    def body(h_hbm, rec_hbm, zrow_hbm, ones_hbm,
             acc_out, acc_sh, rec_v, q_v, rows_v, sem):
        cid = lax.axis_index("c")
        sid = lax.axis_index("s")
        cstart = sid * NCHUNK
        slab0 = sid * SLAB

        for p in range(2):
            pltpu.sync_copy(zrow_hbm, rows_v)
            for k in range(SLAB // WCH):
                r0 = slab0 + k * WCH
                pltpu.sync_copy(rows_v, acc_sh.at[pl.ds(r0, WCH), :])
            plsc.subcore_barrier()
            if ones_mode:
                pltpu.sync_copy(ones_hbm, rows_v)

            cidv = jnp.full((LANES,), cid, jnp.int32)
            onev = jnp.full((LANES,), 1, jnp.int32)
            twov = jnp.full((LANES,), 2, jnp.int32)
            threev = jnp.full((LANES,), 3, jnp.int32)
            planev = jnp.full((LANES,), PLANE, jnp.int32)
            dummyv = jnp.full((LANES,), DUMMY, jnp.int32)
            lov = jnp.full((LANES,), p * HALF, jnp.int32)
            hiv = jnp.full((LANES,), (p + 1) * HALF, jnp.int32)

            def chunk(i, carry):
                base = pl.multiple_of((cstart + i) * 2 * EC, 8)
                pltpu.sync_copy(rec_hbm.at[pl.ds(base, 2 * EC)], rec_v)
                if not ones_mode:
                    pltpu.async_copy(h_hbm.at[rec_v.at[pl.ds(0, EC)]],
                                     rows_v, sem).wait()
                for j in range(EC // LANES):
                    k16 = rec_v[pl.ds(EC + j * LANES, LANES)]
                    t16 = jnp.bitwise_and(k16, threev)
                    d16 = lax.shift_right_logical(k16, twov)
                    plane_t = lax.shift_right_logical(t16, onev)
                    low = jnp.bitwise_and(t16, onev)
                    owned = (plane_t == cidv) & (d16 >= lov) & (d16 < hiv)
                    q = jnp.where(owned, low * planev + (d16 - lov), dummyv)
                    q_v[pl.ds(j * LANES, LANES)] = q
                pltpu.sync_copy(rows_v, acc_sh.at[q_v], add=True)
                return carry

            lax.fori_loop(0, NCHUNK, chunk, 0)
            plsc.subcore_barrier()

            obase = (cid * 2 + p) * SPR
            for k in range(SLAB // WCH):
                r0 = slab0 + k * WCH
                pltpu.sync_copy(acc_sh.at[pl.ds(r0, WCH), :], rows_v)
                pltpu.sync_copy(rows_v,
                                acc_out.at[pl.ds(obase + r0, WCH), :])
    return pl.kernel(
        body,
        out_type=out_type,
        mesh=plsc.VectorSubcoreMesh(**_SC_MESH),
        scratch_types=scratch,
        name="sc_count" if ones_mode else "sc_scatter",
    )


_sc_scatter_fn = functools.lru_cache(maxsize=None)(_make_sc_scatter)


def _embed_body(nt_ref, emb_ref, out_ref):
    onehot = (nt_ref[...] == lax.broadcasted_iota(jnp.int32, (1, V), 1))
    out_ref[...] = jnp.dot(onehot.astype(jnp.float32), emb_ref[...],
                           preferred_element_type=jnp.float32)


_embed = pl.pallas_call(
    _embed_body,
    grid=(N // RB,),
    in_specs=[
        pl.BlockSpec((RB, 1), lambda i: (i, 0)),
        pl.BlockSpec((V, D), lambda i: (0, 0)),
    ],
    out_specs=pl.BlockSpec((RB, D), lambda i: (i, 0)),
    out_shape=jax.ShapeDtypeStruct((N, D), jnp.float32),
)


def _dotT(a, b):
    return lax.dot_general(a, b, (((1,), (1,)), ((), ())),
                           preferred_element_type=jnp.float32)


def _round_body(h_ref, acc_ref, cnt_ref, mw_ref, mb_ref, wih_ref, whh_ref,
                bih_ref, bhh_ref, out_ref):
    h = h_ref[...]
    total = jnp.zeros((RB, MH), jnp.float32)
    for c in range(NC):
        for pz in range(2):
            t = 2 * c + pz
            total = total + _dotT(acc_ref[c, 0, pz], mw_ref[t])
            total = total + cnt_ref[c, 0, pz][:, 0:1] * mb_ref[t]
    msgs = jnp.maximum(total, 0.0)
    gi = _dotT(msgs, wih_ref[...]) + bih_ref[...]
    gh = _dotT(h, whh_ref[...]) + bhh_ref[...]
    r = jax.nn.sigmoid(gi[:, :D] + gh[:, :D])
    z = jax.nn.sigmoid(gi[:, D:2 * D] + gh[:, D:2 * D])
    cbar = jnp.tanh(gi[:, 2 * D:] + r * gh[:, 2 * D:])
    out_ref[...] = (1.0 - z) * cbar + z * h


_round_tc = pl.pallas_call(
    _round_body,
    grid=(N // RB,),
    in_specs=[
        pl.BlockSpec((RB, D), lambda i: (i, 0)),
        pl.BlockSpec((NC, 1, 2, RB, D), lambda i: (0, i // 5, 0, i % 5, 0)),
        pl.BlockSpec((NC, 1, 2, RB, D), lambda i: (0, i // 5, 0, i % 5, 0)),
        pl.BlockSpec((NT, MH, D), lambda i: (0, 0, 0)),
        pl.BlockSpec((NT, 1, MH), lambda i: (0, 0, 0)),
        pl.BlockSpec((3 * D, MH), lambda i: (0, 0)),
        pl.BlockSpec((3 * D, D), lambda i: (0, 0)),
        pl.BlockSpec((1, 3 * D), lambda i: (0, 0)),
        pl.BlockSpec((1, 3 * D), lambda i: (0, 0)),
    ],
    out_specs=pl.BlockSpec((RB, D), lambda i: (i, 0)),
    out_shape=jax.ShapeDtypeStruct((N, D), jnp.float32),
)


def _readout_body(h_ref, ptr_ref, gw_ref, gb_ref, nw_ref, nb_ref, lw_ref,
                  lb_ref, vw_ref, vb_ref, zl_ref, zv_ref, hg):
    i = pl.program_id(0)
    h = h_ref[...]
    attn = jax.nn.sigmoid(_dotT(gw_ref[...], h) + gb_ref[...])
    h2 = _dotT(h, nw_ref[...]) + nb_ref[...]
    g = i * RB + lax.broadcasted_iota(jnp.int32, (1, RB), 1)
    seg = jnp.sum((ptr_ref[...] <= g).astype(jnp.int32), axis=0,
                  keepdims=True)
    oneT = (lax.broadcasted_iota(jnp.int32, (V, 1), 0) == seg)
    aw = oneT.astype(jnp.float32) * attn
    contrib = jnp.dot(aw, h2, preferred_element_type=jnp.float32)

    @pl.when(i == 0)
    def _():
        hg[...] = jnp.zeros_like(hg)

    hg[...] += contrib

    @pl.when(i == pl.num_programs(0) - 1)
    def _():
        zl = _dotT(hg[...], lw_ref[...]) + lb_ref[...]
        zv = _dotT(hg[...], vw_ref[...]) + vb_ref[...]
        zl_ref[...] = zl[:B, :]
        zv_ref[...] = zv[:B, :]


_readout = pl.pallas_call(
    _readout_body,
    grid=(N // RB,),
    in_specs=[
        pl.BlockSpec((RB, D), lambda i: (i, 0)),
        pl.BlockSpec((V, 1), lambda i: (0, 0)),
        pl.BlockSpec((1, D), lambda i: (0, 0)),
        pl.BlockSpec((1, RB), lambda i: (0, 0)),
        pl.BlockSpec((G, D), lambda i: (0, 0)),
        pl.BlockSpec((1, G), lambda i: (0, 0)),
        pl.BlockSpec((LT, G), lambda i: (0, 0)),
        pl.BlockSpec((1, LT), lambda i: (0, 0)),
        pl.BlockSpec((LT, G), lambda i: (0, 0)),
        pl.BlockSpec((1, LT), lambda i: (0, 0)),
    ],
    out_specs=[
        pl.BlockSpec((B, LT), lambda i: (0, 0)),
        pl.BlockSpec((B, LT), lambda i: (0, 0)),
    ],
    out_shape=[
        jax.ShapeDtypeStruct((B, LT), jnp.float32),
        jax.ShapeDtypeStruct((B, LT), jnp.float32),
    ],
    scratch_shapes=[pltpu.VMEM((V, G), jnp.float32)],
)


def kernel(node_types, edge_index, edge_type, ptr, emb_table, msg_W, msg_b,
           gru_w_ih, gru_w_hh, gru_b_ih, gru_b_hh, gate_w, gate_b, n2g_w,
           n2g_b, loc_w, loc_b, logvar_w, logvar_b):
    nt_col = node_types.astype(jnp.int32).reshape(N, 1)
    dst = edge_index[0].astype(jnp.int32)
    src = edge_index[1].astype(jnp.int32)
    typ = edge_type.astype(jnp.int32)
    ptr_pad = jnp.concatenate(
        [ptr[1:B].astype(jnp.int32),
         jnp.full((V - (B - 1),), N, jnp.int32)]).reshape(V, 1)
    mb3 = msg_b.reshape(NT, 1, MH)
    bih = gru_b_ih.reshape(1, 3 * D)
    bhh = gru_b_hh.reshape(1, 3 * D)
    gb = jnp.full((1, RB), gate_b[0], jnp.float32)
    nb = n2g_b.reshape(1, G)
    lb = loc_b.reshape(1, LT)
    vb = logvar_b.reshape(1, LT)
    zrow = jnp.zeros((WCH, D), jnp.float32)
    ones = jnp.ones((EC, D), jnp.float32)
    rec = jnp.concatenate([src.reshape(-1, EC),
                           (dst * 4 + typ).reshape(-1, EC)],
                          axis=1).reshape(-1)

    h = _embed(nt_col, emb_table)
    cnt = _sc_scatter_fn(True)(h, rec, zrow, ones)
    cnt5 = cnt.reshape(NC, 2, 2, PLANE, D)
    for rnd in range(T):
        acc = _sc_scatter_fn(False)(h, rec, zrow, ones)
        acc5 = acc.reshape(NC, 2, 2, PLANE, D)
        h = _round_tc(h, acc5, cnt5, msg_W, mb3, gru_w_ih, gru_w_hh, bih, bhh)
    return _readout(h, ptr_pad, gate_w, gb, n2g_w, nb, loc_w, lb,
                    logvar_w, vb)


# speedup vs baseline: 9.7823x; 1.4164x over previous
import functools

import jax
import jax.numpy as jnp
from jax import lax
from jax.experimental import pallas as pl
from jax.experimental.pallas import tpu as pltpu
from jax.experimental.pallas import tpu_sc as plsc

N = 10000
E = 320000
D = 128
MH = 2 * D
T = 3
NT = 4
G = 256
LT = 64
V = 128
B = 100

NC, NS, LANES = 2, 16, 16
HALF = N // 2
PLANE = 5120
SPR = 2 * PLANE
SLAB = SPR // NS
WCH = 80
EC = 80
EPT = E // NS
NCHUNK = EPT // EC
DUMMY = HALF
RB = 1000

_SC_MESH = dict(core_axis_name="c", subcore_axis_name="s",
                num_cores=NC, num_subcores=NS)


def _make_sc_scatter(ones_mode):
    out_type = jax.ShapeDtypeStruct((NC * 2 * SPR, D), jnp.float32)
    scratch = [
        pltpu.VMEM_SHARED((SPR, D), jnp.float32),
        pltpu.VMEM((2 * EC,), jnp.int32),
        pltpu.VMEM((2 * EC,), jnp.int32),
        pltpu.VMEM((EC,), jnp.int32),
        pltpu.VMEM((EC, D), jnp.float32),
        pltpu.VMEM((EC, D), jnp.float32),
        pltpu.VMEM((WCH, D), jnp.float32),
        pltpu.SemaphoreType.DMA,
        pltpu.SemaphoreType.DMA,
        pltpu.SemaphoreType.DMA,
        pltpu.SemaphoreType.DMA,
    ]

    def body(h_hbm, rec_hbm, zrow_hbm, ones_hbm,
             acc_out, acc_sh, rA, rB, q_v, rowsA, rowsB, wbuf,
             gsA, gsB, rsA, rsB):
        cid = lax.axis_index("c")
        sid = lax.axis_index("s")
        cstart = sid * NCHUNK
        slab0 = sid * SLAB
        REC = 2 * EC

        def rec_slice(ci):
            return rec_hbm.at[pl.ds(pl.multiple_of(ci * REC, 8), REC)]

        for p in range(2):
            pltpu.sync_copy(zrow_hbm, wbuf)
            for k in range(SLAB // WCH):
                r0 = slab0 + k * WCH
                pltpu.sync_copy(wbuf, acc_sh.at[pl.ds(r0, WCH), :])
            plsc.subcore_barrier()
            if ones_mode:
                pltpu.sync_copy(ones_hbm, rowsA)

            cidv = jnp.full((LANES,), cid, jnp.int32)
            onev = jnp.full((LANES,), 1, jnp.int32)
            twov = jnp.full((LANES,), 2, jnp.int32)
            threev = jnp.full((LANES,), 3, jnp.int32)
            planev = jnp.full((LANES,), PLANE, jnp.int32)
            dummyv = jnp.full((LANES,), DUMMY, jnp.int32)
            lov = jnp.full((LANES,), p * HALF, jnp.int32)
            hiv = jnp.full((LANES,), (p + 1) * HALF, jnp.int32)

            def q_from(rbuf):
                for j in range(EC // LANES):
                    k16 = rbuf[pl.ds(EC + j * LANES, LANES)]
                    t16 = jnp.bitwise_and(k16, threev)
                    d16 = lax.shift_right_logical(k16, twov)
                    plane_t = lax.shift_right_logical(t16, onev)
                    low = jnp.bitwise_and(t16, onev)
                    owned = (plane_t == cidv) & (d16 >= lov) & (d16 < hiv)
                    q = jnp.where(owned, low * planev + (d16 - lov), dummyv)
                    q_v[pl.ds(j * LANES, LANES)] = q

            pltpu.sync_copy(rec_slice(cstart), rA)
            if not ones_mode:
                pltpu.async_copy(h_hbm.at[rA.at[pl.ds(0, EC)]], rowsA, gsA)
            pltpu.async_copy(rec_slice(cstart + 1), rB, rsB)

            def dbl(k, carry):
                i = 2 * k
                if not ones_mode:
                    pltpu.make_async_copy(
                        h_hbm.at[rA.at[pl.ds(0, EC)]], rowsA, gsA).wait()
                q_from(rA)
                pltpu.async_copy(rec_slice(cstart + i + 2), rA, rsA)
                pltpu.make_async_copy(rec_slice(0), rB, rsB).wait()
                if not ones_mode:
                    pltpu.async_copy(h_hbm.at[rB.at[pl.ds(0, EC)]],
                                    rowsB, gsB)
                pltpu.sync_copy(rowsA, acc_sh.at[q_v], add=True)
                if not ones_mode:
                    pltpu.make_async_copy(
                        h_hbm.at[rB.at[pl.ds(0, EC)]], rowsB, gsB).wait()
                q_from(rB)
                pltpu.async_copy(rec_slice(cstart + i + 3), rB, rsB)
                pltpu.make_async_copy(rec_slice(0), rA, rsA).wait()
                if not ones_mode:
                    pltpu.async_copy(h_hbm.at[rA.at[pl.ds(0, EC)]],
                                    rowsA, gsA)
                pltpu.sync_copy(rowsB if not ones_mode else rowsA,
                                acc_sh.at[q_v], add=True)
                return carry

            lax.fori_loop(0, NCHUNK // 2, dbl, 0)
            if not ones_mode:
                pltpu.make_async_copy(
                    h_hbm.at[rA.at[pl.ds(0, EC)]], rowsA, gsA).wait()
            pltpu.make_async_copy(rec_slice(0), rB, rsB).wait()
            plsc.subcore_barrier()

            obase = (cid * 2 + p) * SPR
            for k in range(SLAB // WCH):
                r0 = slab0 + k * WCH
                pltpu.sync_copy(acc_sh.at[pl.ds(r0, WCH), :], wbuf)
                pltpu.sync_copy(wbuf,
                                acc_out.at[pl.ds(obase + r0, WCH), :])

    return pl.kernel(
        body,
        out_type=out_type,
        mesh=plsc.VectorSubcoreMesh(**_SC_MESH),
        scratch_types=scratch,
        name="sc_count" if ones_mode else "sc_scatter",
    )


_sc_scatter_fn = functools.lru_cache(maxsize=None)(_make_sc_scatter)


def _embed_body(nt_ref, emb_ref, out_ref):
    onehot = (nt_ref[...] == lax.broadcasted_iota(jnp.int32, (1, V), 1))
    out_ref[...] = jnp.dot(onehot.astype(jnp.float32), emb_ref[...],
                           preferred_element_type=jnp.float32)


_embed = pl.pallas_call(
    _embed_body,
    grid=(N // RB,),
    in_specs=[
        pl.BlockSpec((RB, 1), lambda i: (i, 0)),
        pl.BlockSpec((V, D), lambda i: (0, 0)),
    ],
    out_specs=pl.BlockSpec((RB, D), lambda i: (i, 0)),
    out_shape=jax.ShapeDtypeStruct((N, D), jnp.float32),
)


def _dotT(a, b):
    return lax.dot_general(a, b, (((1,), (1,)), ((), ())),
                           preferred_element_type=jnp.float32)


def _round_body(h_ref, acc_ref, cnt_ref, mw_ref, mb_ref, wih_ref, whh_ref,
                bih_ref, bhh_ref, out_ref):
    h = h_ref[...]
    total = jnp.zeros((RB, MH), jnp.float32)
    for c in range(NC):
        for pz in range(2):
            t = 2 * c + pz
            total = total + _dotT(acc_ref[c, 0, pz], mw_ref[t])
            total = total + cnt_ref[c, 0, pz][:, 0:1] * mb_ref[t]
    msgs = jnp.maximum(total, 0.0)
    gi = _dotT(msgs, wih_ref[...]) + bih_ref[...]
    gh = _dotT(h, whh_ref[...]) + bhh_ref[...]
    r = jax.nn.sigmoid(gi[:, :D] + gh[:, :D])
    z = jax.nn.sigmoid(gi[:, D:2 * D] + gh[:, D:2 * D])
    cbar = jnp.tanh(gi[:, 2 * D:] + r * gh[:, 2 * D:])
    out_ref[...] = (1.0 - z) * cbar + z * h


_round_tc = pl.pallas_call(
    _round_body,
    grid=(N // RB,),
    in_specs=[
        pl.BlockSpec((RB, D), lambda i: (i, 0)),
        pl.BlockSpec((NC, 1, 2, RB, D), lambda i: (0, i // 5, 0, i % 5, 0)),
        pl.BlockSpec((NC, 1, 2, RB, D), lambda i: (0, i // 5, 0, i % 5, 0)),
        pl.BlockSpec((NT, MH, D), lambda i: (0, 0, 0)),
        pl.BlockSpec((NT, 1, MH), lambda i: (0, 0, 0)),
        pl.BlockSpec((3 * D, MH), lambda i: (0, 0)),
        pl.BlockSpec((3 * D, D), lambda i: (0, 0)),
        pl.BlockSpec((1, 3 * D), lambda i: (0, 0)),
        pl.BlockSpec((1, 3 * D), lambda i: (0, 0)),
    ],
    out_specs=pl.BlockSpec((RB, D), lambda i: (i, 0)),
    out_shape=jax.ShapeDtypeStruct((N, D), jnp.float32),
)


def _readout_body(h_ref, ptr_ref, gw_ref, gb_ref, nw_ref, nb_ref, lw_ref,
                  lb_ref, vw_ref, vb_ref, zl_ref, zv_ref, hg):
    i = pl.program_id(0)
    h = h_ref[...]
    attn = jax.nn.sigmoid(_dotT(gw_ref[...], h) + gb_ref[...])
    h2 = _dotT(h, nw_ref[...]) + nb_ref[...]
    g = i * RB + lax.broadcasted_iota(jnp.int32, (1, RB), 1)
    seg = jnp.sum((ptr_ref[...] <= g).astype(jnp.int32), axis=0,
                  keepdims=True)
    oneT = (lax.broadcasted_iota(jnp.int32, (V, 1), 0) == seg)
    aw = oneT.astype(jnp.float32) * attn
    contrib = jnp.dot(aw, h2, preferred_element_type=jnp.float32)

    @pl.when(i == 0)
    def _():
        hg[...] = jnp.zeros_like(hg)

    hg[...] += contrib

    @pl.when(i == pl.num_programs(0) - 1)
    def _():
        zl = _dotT(hg[...], lw_ref[...]) + lb_ref[...]
        zv = _dotT(hg[...], vw_ref[...]) + vb_ref[...]
        zl_ref[...] = zl[:B, :]
        zv_ref[...] = zv[:B, :]


_readout = pl.pallas_call(
    _readout_body,
    grid=(N // RB,),
    in_specs=[
        pl.BlockSpec((RB, D), lambda i: (i, 0)),
        pl.BlockSpec((V, 1), lambda i: (0, 0)),
        pl.BlockSpec((1, D), lambda i: (0, 0)),
        pl.BlockSpec((1, RB), lambda i: (0, 0)),
        pl.BlockSpec((G, D), lambda i: (0, 0)),
        pl.BlockSpec((1, G), lambda i: (0, 0)),
        pl.BlockSpec((LT, G), lambda i: (0, 0)),
        pl.BlockSpec((1, LT), lambda i: (0, 0)),
        pl.BlockSpec((LT, G), lambda i: (0, 0)),
        pl.BlockSpec((1, LT), lambda i: (0, 0)),
    ],
    out_specs=[
        pl.BlockSpec((B, LT), lambda i: (0, 0)),
        pl.BlockSpec((B, LT), lambda i: (0, 0)),
    ],
    out_shape=[
        jax.ShapeDtypeStruct((B, LT), jnp.float32),
        jax.ShapeDtypeStruct((B, LT), jnp.float32),
    ],
    scratch_shapes=[pltpu.VMEM((V, G), jnp.float32)],
)


def kernel(node_types, edge_index, edge_type, ptr, emb_table, msg_W, msg_b,
           gru_w_ih, gru_w_hh, gru_b_ih, gru_b_hh, gate_w, gate_b, n2g_w,
           n2g_b, loc_w, loc_b, logvar_w, logvar_b):
    nt_col = node_types.astype(jnp.int32).reshape(N, 1)
    dst = edge_index[0].astype(jnp.int32)
    src = edge_index[1].astype(jnp.int32)
    typ = edge_type.astype(jnp.int32)
    ptr_pad = jnp.concatenate(
        [ptr[1:B].astype(jnp.int32),
         jnp.full((V - (B - 1),), N, jnp.int32)]).reshape(V, 1)
    mb3 = msg_b.reshape(NT, 1, MH)
    bih = gru_b_ih.reshape(1, 3 * D)
    bhh = gru_b_hh.reshape(1, 3 * D)
    gb = jnp.full((1, RB), gate_b[0], jnp.float32)
    nb = n2g_b.reshape(1, G)
    lb = loc_b.reshape(1, LT)
    vb = logvar_b.reshape(1, LT)
    zrow = jnp.zeros((WCH, D), jnp.float32)
    ones = jnp.ones((EC, D), jnp.float32)
    rec = jnp.concatenate([src.reshape(-1, EC),
                           (dst * 4 + typ).reshape(-1, EC)],
                          axis=1).reshape(-1)
    rec = jnp.concatenate([rec, jnp.zeros((4 * EC,), jnp.int32)])

    h = _embed(nt_col, emb_table)
    cnt = _sc_scatter_fn(True)(h, rec, zrow, ones)
    cnt5 = cnt.reshape(NC, 2, 2, PLANE, D)
    for rnd in range(T):
        acc = _sc_scatter_fn(False)(h, rec, zrow, ones)
        acc5 = acc.reshape(NC, 2, 2, PLANE, D)
        h = _round_tc(h, acc5, cnt5, msg_W, mb3, gru_w_ih, gru_w_hh, bih, bhh)
    return _readout(h, ptr_pad, gate_w, gb, n2g_w, nb, loc_w, lb,
                    logvar_w, vb)
